# baseline (speedup 1.0000x reference)
import functools

import jax
import jax.numpy as jnp
from jax import lax
from jax.experimental import pallas as pl
from jax.experimental.pallas import tpu as pltpu
from jax.experimental.pallas import tpu_sc as plsc

B, T, D = 4, 2048, 64
NC, NS = 2, 16
NW = NC * NS
N = B * T
RPW = N // NW
IDX_MINOR = 128
IDX_ROWS = RPW // IDX_MINOR


def _emb_body(x_hbm, tok_hbm, pos_hbm, out_hbm, idx_v, rows_v, pos_v, sem):
    c = lax.axis_index("c")
    s = lax.axis_index("s")
    wid = s * NC + c
    base = wid * RPW
    pos_base = lax.rem(base, T)

    pltpu.sync_copy(x_hbm.at[pl.ds(wid * IDX_ROWS, IDX_ROWS)], idx_v)
    copies = [
        pltpu.async_copy(
            tok_hbm.at[idx_v.at[j]],
            rows_v.at[pl.ds(j * IDX_MINOR, IDX_MINOR)],
            sem,
        )
        for j in range(IDX_ROWS)
    ]
    pltpu.sync_copy(pos_hbm.at[pl.ds(pos_base, RPW)], pos_v)
    for cp in copies:
        cp.wait()

    def add_row(i, carry):
        for j in range(D // 16):
            sl = (i, pl.ds(j * 16, 16))
            rows_v[sl] = rows_v[sl] + pos_v[sl]
        return carry

    lax.fori_loop(0, RPW, add_row, None)
    pltpu.sync_copy(rows_v, out_hbm.at[pl.ds(base, RPW)])


@functools.partial(jax.jit, static_argnames=())
def _emb(x2d, tok, pos):
    kfn = pl.kernel(
        _emb_body,
        out_type=jax.ShapeDtypeStruct((N, D), jnp.float32),
        mesh=plsc.VectorSubcoreMesh(core_axis_name="c", subcore_axis_name="s"),
        scratch_types=[
            pltpu.VMEM((IDX_ROWS, IDX_MINOR), jnp.int32),
            pltpu.VMEM((RPW, D), jnp.float32),
            pltpu.VMEM((RPW, D), jnp.float32),
            pltpu.SemaphoreType.DMA,
        ],
        compiler_params=pltpu.CompilerParams(use_tc_tiling_on_sc=False),
    )
    return kfn(x2d, tok, pos)


def kernel(x, token_table, pos_table):
    x2d = x.astype(jnp.int32).reshape(NW * IDX_ROWS, IDX_MINOR)
    out = _emb(x2d, token_table, pos_table)
    return out.reshape(B, T, D)


# speedup vs baseline: 4.4547x; 4.4547x over previous
import jax
import jax.numpy as jnp
from jax import lax
from jax.experimental import pallas as pl
from jax.experimental.pallas import tpu as pltpu
from jax.experimental.pallas import tpu_sc as plsc

B, T, D = 4, 2048, 64
NC, NS = 2, 16
NW = NC * NS
N = B * T
RPW = N // NW
WPB = T // RPW
WIN = 128
G = 4
NG = RPW // G
NSLOT = 2 * G


def _emb_body(x_hbm, tokT_hbm, posT_hbm, out_hbm, idx_v, win_v, acc_v,
              pos_v, sem_a, sem_b):
    c = lax.axis_index("c")
    s = lax.axis_index("s")
    wid = s * NC + c
    b = wid // WPB
    t0 = (wid % WPB) * RPW

    pltpu.sync_copy(x_hbm.at[b, pl.ds(t0, RPW)], idx_v)
    pcp = pltpu.async_copy(posT_hbm.at[:, pl.ds(t0, RPW)], pos_v, sem_b)

    lane16 = lax.iota(jnp.int32, 16)

    def scal_idx(j):
        chunk = idx_v[pl.ds(pl.multiple_of((j // 16) * 16, 16), 16)]
        m = lane16 == jnp.full((16,), j % 16, jnp.int32)
        return jnp.max(jnp.where(m, chunk, 0))

    def group_fire(g, sem, slot0):
        for u in range(G):
            i = scal_idx(g * G + u)
            base = pl.multiple_of((i // WIN) * WIN, WIN)
            pltpu.async_copy(
                tokT_hbm.at[:, pl.ds(base, WIN)], win_v.at[slot0 + u], sem
            )

    def group_drain(sem, slot0):
        pltpu.make_async_copy(
            tokT_hbm.at[:, pl.ds(0, G * WIN)],
            win_v.at[pl.ds(slot0, G)],
            sem,
        ).wait()

    def group_extract(g, slot0):
        for u in range(G):
            j = g * G + u
            lam = jnp.full((16,), jnp.remainder(scal_idx(j), WIN), jnp.int32)
            j16 = jnp.full((16,), j, jnp.int32)
            for dc in range(D // 16):
                d16 = dc * 16 + lane16
                vals = plsc.load_gather(win_v.at[slot0 + u], [d16, lam])
                plsc.store_scatter(acc_v, [d16, j16], vals)

    pcp.wait()
    group_fire(0, sem_a, 0)

    def step(k, carry):
        g0 = 2 * k
        group_fire(g0 + 1, sem_b, G)
        group_drain(sem_a, 0)
        group_extract(g0, 0)

        @pl.when(k < NG // 2 - 1)
        def _():
            group_fire(g0 + 2, sem_a, 0)

        group_drain(sem_b, G)
        group_extract(g0 + 1, G)
        return carry

    lax.fori_loop(0, NG // 2, step, None)

    def add_row(d, carry):
        for tb in range(RPW // 16):
            sl = (d, pl.ds(tb * 16, 16))
            acc_v[sl] = acc_v[sl] + pos_v[sl]
        return carry

    lax.fori_loop(0, D, add_row, None)
    pltpu.sync_copy(acc_v, out_hbm.at[b, :, pl.ds(t0, RPW)])


@jax.jit
def _emb(x, tokT, posT):
    kfn = pl.kernel(
        _emb_body,
        out_type=jax.ShapeDtypeStruct((B, D, T), jnp.float32),
        mesh=plsc.VectorSubcoreMesh(core_axis_name="c", subcore_axis_name="s"),
        scratch_types=[
            pltpu.VMEM((RPW,), jnp.int32),
            pltpu.VMEM((NSLOT, D, WIN), jnp.float32),
            pltpu.VMEM((D, RPW), jnp.float32),
            pltpu.VMEM((D, RPW), jnp.float32),
            pltpu.SemaphoreType.DMA,
            pltpu.SemaphoreType.DMA,
        ],
        compiler_params=pltpu.CompilerParams(
            use_tc_tiling_on_sc=True, needs_layout_passes=False
        ),
    )
    return kfn(x, tokT, posT)


def kernel(x, token_table, pos_table):
    out = _emb(x.astype(jnp.int32), token_table.T, pos_table.T)
    return out.transpose(0, 2, 1)


# speedup vs baseline: 4.5164x; 1.0138x over previous
import jax
import jax.numpy as jnp
from jax import lax
from jax.experimental import pallas as pl
from jax.experimental.pallas import tpu as pltpu
from jax.experimental.pallas import tpu_sc as plsc

B, T, D = 4, 2048, 64
NC, NS = 2, 16
NW = NC * NS
N = B * T
RPW = N // NW
WPB = T // RPW
WIN = 128
G = 4
NG = RPW // G
NSLOT = 2 * G


def _emb_body(x_hbm, tokT_hbm, posT_hbm, out_hbm, idx_v, win_v, acc_v,
              pos_v, sem_a, sem_b, sem_p):
    c = lax.axis_index("c")
    s = lax.axis_index("s")
    wid = s * NC + c
    b = wid // WPB
    t0 = (wid % WPB) * RPW

    pltpu.sync_copy(x_hbm.at[b, pl.ds(t0, RPW)], idx_v)
    pcp = pltpu.async_copy(posT_hbm.at[:, pl.ds(t0, RPW)], pos_v, sem_p)

    lane16 = lax.iota(jnp.int32, 16)

    def scal_idx(j):
        chunk = idx_v[pl.ds(pl.multiple_of((j // 16) * 16, 16), 16)]
        m = lane16 == jnp.full((16,), j % 16, jnp.int32)
        return jnp.max(jnp.where(m, chunk, 0))

    def group_fire(g, sem, slot0):
        for u in range(G):
            i = scal_idx(g * G + u)
            base = pl.multiple_of((i // WIN) * WIN, WIN)
            pltpu.async_copy(
                tokT_hbm.at[:, pl.ds(base, WIN)], win_v.at[slot0 + u], sem
            )

    def group_drain(sem, slot0):
        pltpu.make_async_copy(
            tokT_hbm.at[:, pl.ds(0, G * WIN)],
            win_v.at[pl.ds(slot0, G)],
            sem,
        ).wait()

    def group_extract(g, slot0):
        for u in range(G):
            j = g * G + u
            lam = jnp.full((16,), jnp.remainder(scal_idx(j), WIN), jnp.int32)
            j16 = jnp.full((16,), j, jnp.int32)
            for dc in range(D // 16):
                d16 = dc * 16 + lane16
                vals = plsc.load_gather(win_v.at[slot0 + u], [d16, lam])
                plsc.store_scatter(acc_v, [d16, j16], vals)

    group_fire(0, sem_a, 0)

    def step(k, carry):
        g0 = 2 * k
        group_fire(g0 + 1, sem_b, G)
        group_drain(sem_a, 0)
        group_extract(g0, 0)

        @pl.when(k < NG // 2 - 1)
        def _():
            group_fire(g0 + 2, sem_a, 0)

        group_drain(sem_b, G)
        group_extract(g0 + 1, G)
        return carry

    lax.fori_loop(0, NG // 2, step, None)
    pcp.wait()

    def add_row(d, carry):
        for tb in range(RPW // 16):
            sl = (d, pl.ds(tb * 16, 16))
            acc_v[sl] = acc_v[sl] + pos_v[sl]
        return carry

    lax.fori_loop(0, D, add_row, None)
    pltpu.sync_copy(acc_v, out_hbm.at[b, :, pl.ds(t0, RPW)])


@jax.jit
def _emb(x, tokT, posT):
    kfn = pl.kernel(
        _emb_body,
        out_type=jax.ShapeDtypeStruct((B, D, T), jnp.float32),
        mesh=plsc.VectorSubcoreMesh(core_axis_name="c", subcore_axis_name="s"),
        scratch_types=[
            pltpu.VMEM((RPW,), jnp.int32),
            pltpu.VMEM((NSLOT, D, WIN), jnp.float32),
            pltpu.VMEM((D, RPW), jnp.float32),
            pltpu.VMEM((D, RPW), jnp.float32),
            pltpu.SemaphoreType.DMA,
            pltpu.SemaphoreType.DMA,
            pltpu.SemaphoreType.DMA,
        ],
        compiler_params=pltpu.CompilerParams(
            use_tc_tiling_on_sc=True, needs_layout_passes=False
        ),
    )
    return kfn(x, tokT, posT)


def kernel(x, token_table, pos_table):
    out = _emb(x.astype(jnp.int32), token_table.T, pos_table.T)
    return out.transpose(0, 2, 1)
